# baseline (speedup 1.0000x reference)
import jax
import jax.numpy as jnp
from jax.experimental import pallas as pl
from jax.experimental.pallas import tpu as pltpu

_MAX_DELAY = 8
_TILE = 2048
_CHUNK = 512
_NCH = _TILE // _CHUNK
_NXB = 8
_K = 6
_NSEM = 8


def _in_copy(x_hbm, xbuf, in_sems, c, n_b, n_chunks_per_b):
    step = c // _NCH
    h = jax.lax.rem(c, _NCH)
    t = step // n_b
    b = jax.lax.rem(step, n_b)
    row0 = t * _TILE + h * _CHUNK
    slot = jax.lax.rem(c, _NXB)
    del n_chunks_per_b
    return pltpu.make_async_copy(
        x_hbm.at[b, pl.ds(row0, _CHUNK), :], xbuf.at[slot], in_sems.at[slot]
    )


def _body(delay_ref, x_hbm, pe_ref, off_ref, out_ref, xbuf, oscr, in_sems, out_sems):
    del delay_ref
    t = pl.program_id(0)
    b = pl.program_id(1)
    n_b = pl.num_programs(1)
    n_t = pl.num_programs(0)
    step = t * n_b + b
    total_chunks = n_t * n_b * _NCH

    @pl.when(step == 0)
    def _prologue():
        for c0 in range(_K):
            _in_copy(x_hbm, xbuf, in_sems, jnp.int32(c0), n_b, None).start()

    for h in range(_NCH):
        c = _NCH * step + h
        row0 = t * _TILE + h * _CHUNK
        islot = jax.lax.rem(c, _NXB)
        oslot = jax.lax.rem(c, _NSEM)

        @pl.when(c + _K < total_chunks)
        def _issue_ahead():
            _in_copy(x_hbm, xbuf, in_sems, c + _K, n_b, None).start()

        _in_copy(x_hbm, xbuf, in_sems, c, n_b, None).wait()

        @pl.when(c >= _NSEM)
        def _wait_out_slot():
            pltpu.make_async_copy(
                oscr.at[oslot], out_ref.at[b, pl.ds(row0, _CHUNK), :], out_sems.at[oslot]
            ).wait()

        oscr[oslot] = (
            xbuf[islot]
            + pe_ref[h * _CHUNK : (h + 1) * _CHUNK]
            + off_ref[0]
        )
        pltpu.make_async_copy(
            oscr.at[oslot], out_ref.at[b, pl.ds(row0, _CHUNK), :], out_sems.at[oslot]
        ).start()

    @pl.when(step == n_t * n_b - 1)
    def _drain():
        for k in range(_NSEM):
            pltpu.make_async_copy(
                oscr.at[k], out_ref.at[b, pl.ds(t * _TILE, _CHUNK), :], out_sems.at[k]
            ).wait()


def kernel(x, delay, offset_embed, sin_pe):
    B, T, D = x.shape
    pe = sin_pe[:T]
    off3 = offset_embed.reshape(offset_embed.shape[0], 1, D)
    n_t = T // _TILE

    grid_spec = pltpu.PrefetchScalarGridSpec(
        num_scalar_prefetch=1,
        grid=(n_t, B),
        in_specs=[
            pl.BlockSpec(memory_space=pltpu.MemorySpace.HBM),
            pl.BlockSpec((_TILE, D), lambda t, b, d: (t, 0)),
            pl.BlockSpec((1, 1, D), lambda t, b, d: (jnp.clip(d[b], 0, _MAX_DELAY), 0, 0)),
        ],
        out_specs=pl.BlockSpec(memory_space=pltpu.MemorySpace.HBM),
        scratch_shapes=[
            pltpu.VMEM((_NXB, _CHUNK, D), jnp.float32),
            pltpu.VMEM((_NSEM, _CHUNK, D), jnp.float32),
            pltpu.SemaphoreType.DMA((_NXB,)),
            pltpu.SemaphoreType.DMA((_NSEM,)),
        ],
    )
    return pl.pallas_call(
        _body,
        grid_spec=grid_spec,
        out_shape=jax.ShapeDtypeStruct((B, T, D), x.dtype),
    )(delay, x, pe, off3)
